# Initial kernel scaffold; baseline (speedup 1.0000x reference)
import jax
import jax.numpy as jnp
from jax.experimental import pallas as pl


def kernel(x1, edge_index1, x2, edge_index2, batch, gene_exp, drr, drc, prot_exp, params):
    raise NotImplementedError("write your pallas kernel here")


# speedup vs baseline: 23.6552x; 23.6552x over previous
import functools

import jax
import jax.numpy as jnp
from jax import lax
from jax.experimental import pallas as pl
from jax.experimental.pallas import tpu as pltpu
from jax.experimental.pallas import tpu_sc as plsc

NC, NS, L = 2, 16, 16
NW = NC * NS
F = 64

_MESH = dict(core_axis_name="c", subcore_axis_name="s", num_cores=NC,
             num_subcores=NS)
_CPARAMS = pltpu.CompilerParams(needs_layout_passes=False,
                               internal_scratch_in_bytes=4096)


def _act(x, act):
    if act == "relu":
        return jnp.maximum(x, 0.0)
    if act == "elu":
        return jnp.where(x > 0, x, jnp.exp(jnp.minimum(x, 0.0)) - 1.0)
    return x


def _mm(x, w, b=None, act=None, in_bias=None, in_act=None, l2norm=False,
        bn=512):
    M, K = x.shape
    Nout = w.shape[1]
    bm = M if M <= 2048 else 2000
    gm = pl.cdiv(M, bm)
    bn = min(Nout, bn)
    gn = pl.cdiv(Nout, bn)
    have_b = b is not None
    have_ib = in_bias is not None

    def body(*refs):
        x_ref, w_ref = refs[0], refs[1]
        i = 2
        b_ref = refs[i] if have_b else None
        i += int(have_b)
        ib_ref = refs[i] if have_ib else None
        i += int(have_ib)
        o_ref = refs[i]
        xv = x_ref[...]
        if have_ib:
            xv = xv + ib_ref[...]
        xv = _act(xv, in_act)
        if l2norm:
            nrm = jnp.sqrt(jnp.sum(xv * xv, axis=1, keepdims=True))
            xv = xv / jnp.maximum(nrm, 1e-12)
        acc = jnp.dot(xv, w_ref[...], preferred_element_type=jnp.float32,
                      precision=lax.Precision.HIGHEST)
        if have_b:
            acc = acc + b_ref[...]
        o_ref[...] = _act(acc, act)

    in_specs = [pl.BlockSpec((bm, K), lambda i, j: (i, 0)),
                pl.BlockSpec((K, bn), lambda i, j: (0, j))]
    args = [x, w]
    if have_b:
        in_specs.append(pl.BlockSpec((1, bn), lambda i, j: (0, j)))
        args.append(b.reshape(1, Nout))
    if have_ib:
        in_specs.append(pl.BlockSpec((1, K), lambda i, j: (0, 0)))
        args.append(in_bias.reshape(1, K))
    return pl.pallas_call(
        body,
        grid=(gm, gn),
        in_specs=in_specs,
        out_specs=pl.BlockSpec((bm, bn), lambda i, j: (i, j)),
        out_shape=jax.ShapeDtypeStruct((M, Nout), jnp.float32),
    )(*args)


def _att_logits(h, att_s, att_d):
    Np, HF = h.shape
    H = att_s.shape[0]
    bm = 2000

    def body(h_ref, s_ref, d_ref, o_ref):
        hv = h_ref[...]
        cols = []
        for att_ref in (s_ref, d_ref):
            for k in range(H):
                hk = hv[:, k * F:(k + 1) * F]
                cols.append(jnp.sum(hk * att_ref[k, :][None, :], axis=1))
        o_ref[...] = jnp.stack(cols, axis=1)

    return pl.pallas_call(
        body,
        grid=(Np // bm,),
        in_specs=[pl.BlockSpec((bm, HF), lambda i: (i, 0)),
                  pl.BlockSpec((H, F), lambda i: (0, 0)),
                  pl.BlockSpec((H, F), lambda i: (0, 0))],
        out_specs=pl.BlockSpec((bm, 2 * H), lambda i: (i, 0)),
        out_shape=jax.ShapeDtypeStruct((Np, 2 * H), jnp.float32),
    )(h, att_s, att_d)


def _col_max(lg):
    Np, C = lg.shape
    bm = 2000

    def body(l_ref, o_ref):
        m = jnp.max(l_ref[...], axis=0, keepdims=True)

        @pl.when(pl.program_id(0) == 0)
        def _():
            o_ref[...] = m

        @pl.when(pl.program_id(0) > 0)
        def _():
            o_ref[...] = jnp.maximum(o_ref[...], m)

    return pl.pallas_call(
        body,
        grid=(Np // bm,),
        in_specs=[pl.BlockSpec((bm, C), lambda i: (i, 0))],
        out_specs=pl.BlockSpec((1, C), lambda i: (0, 0)),
        out_shape=jax.ShapeDtypeStruct((1, C), jnp.float32),
    )(lg)


def _sc_coef(eb1, eb2, H, Ereal, Np, NPP):
    EP = eb1[0].shape[0]
    EC = EP // NS
    DRC = H if H == 4 else 1
    HP = 2 if H == 4 else 1
    NB = EC // L
    CHUNK = 1024
    NCH = EC // CHUNK
    NR = NPP // 128
    mesh = plsc.VectorSubcoreMesh(**_MESH)
    sds = jax.ShapeDtypeStruct((DRC, EP), jnp.float32)

    @functools.partial(
        pl.kernel,
        out_type=(sds, sds),
        mesh=mesh,
        compiler_params=_CPARAMS,
        scratch_types=[
            pltpu.VMEM((EC,), jnp.int32),
            pltpu.VMEM((EC,), jnp.int32),
            pltpu.VMEM((NPP,), jnp.float32),
            pltpu.VMEM((NPP,), jnp.float32),
            pltpu.VMEM((NR, 128), jnp.float32),
            pltpu.VMEM((16,), jnp.float32),
            pltpu.VMEM((NR,), jnp.int32),
            pltpu.VMEM((8, 128), jnp.float32),
            pltpu.VMEM((CHUNK,), jnp.float32),
            pltpu.VMEM_SHARED((NR, 128), jnp.float32),
        ],
    )
    def k(s1, d1, l1, v1, s2, d2, l2, v2, o1, o2,
          src_b, dst_b, as_b, ad_b, dn2, dv_b, ridx, zbuf, co_buf, sh_dn):
        c = lax.axis_index("c")
        s = lax.axis_index("s")
        ebase = s * EC
        for r in range(8):
            for f in range(8):
                zbuf[r, pl.ds(f * L, L)] = jnp.zeros((L,), jnp.float32)

        def ri(i, _):
            ridx[pl.ds(i * L, L)] = i * L + lax.iota(jnp.int32, 16)
            return 0

        lax.fori_loop(0, NR // L, ri, 0)

        for (src_hbm, dst_hbm, lgt_hbm, dvec_hbm, co_hbm) in (
                (s1, d1, l1, v1, o1), (s2, d2, l2, v2, o2)):
            pltpu.sync_copy(src_hbm.at[pl.ds(ebase, EC)], src_b)
            pltpu.sync_copy(dst_hbm.at[pl.ds(ebase, EC)], dst_b)
            for kl in range(HP):
                kh = 2 * c + kl if H == 4 else 0
                pltpu.sync_copy(lgt_hbm.at[kh], as_b)
                pltpu.sync_copy(lgt_hbm.at[H + kh], ad_b)
                pltpu.sync_copy(dvec_hbm.at[kh], dv_b)
                dk = dv_b[...]

                def zp(i, _):
                    for f in range(8):
                        dn2[i, pl.ds(f * L, L)] = jnp.zeros((L,),
                                                            jnp.float32)
                    return 0

                lax.fori_loop(0, NR, zp, 0)

                @pl.when(s < NR // 8)
                def _():
                    pltpu.sync_copy(zbuf, sh_dn.at[pl.ds(s * 8, 8)])

                plsc.subcore_barrier()

                def edge_body(i, _):
                    sv = src_b[pl.ds(i * L, L)]
                    dv = dst_b[pl.ds(i * L, L)]
                    asv = plsc.load_gather(as_b, [sv])
                    adv = plsc.load_gather(ad_b, [dv])
                    al = asv + adv
                    al = jnp.maximum(al, 0.2 * al)
                    ex = jnp.exp(al - dk)
                    gid = ebase + i * L + lax.iota(jnp.int32, 16)
                    ex = jnp.where(gid < Ereal, ex, 0.0)
                    lanes = lax.iota(jnp.int32, 16)
                    hi7 = jnp.right_shift(dv, 7)
                    lo7 = jnp.bitwise_and(dv, 127)
                    for lane in range(L):
                        plsc.addupdate_scatter(dn2, [hi7, lo7], ex,
                                               mask=lanes == lane)
                    return 0

                lax.fori_loop(0, NB, edge_body, 0)
                for rr in range(NS):
                    @pl.when(s == rr)
                    def _():
                        pltpu.sync_copy(dn2, sh_dn.at[ridx], add=True)
                    plsc.subcore_barrier()
                pltpu.sync_copy(sh_dn, dn2)

                def rb(i, _):
                    for f in range(8):
                        dn2[i, pl.ds(f * L, L)] = 1.0 / (
                            dn2[i, pl.ds(f * L, L)] + 1e-16)
                    return 0

                lax.fori_loop(0, NR, rb, 0)

                def chunk_body(ch, _):
                    def grp(g, _):
                        off = ch * CHUNK + g * L
                        sv = src_b[pl.ds(off, L)]
                        dv = dst_b[pl.ds(off, L)]
                        asv = plsc.load_gather(as_b, [sv])
                        adv = plsc.load_gather(ad_b, [dv])
                        al = asv + adv
                        al = jnp.maximum(al, 0.2 * al)
                        rdv = plsc.load_gather(
                            dn2, [jnp.right_shift(dv, 7),
                                  jnp.bitwise_and(dv, 127)])
                        co = jnp.exp(al - dk) * rdv
                        gid = ebase + off + lax.iota(jnp.int32, 16)
                        co = jnp.where(gid < Ereal, co, 0.0)
                        co_buf[pl.ds(g * L, L)] = co
                        return 0

                    lax.fori_loop(0, CHUNK // L, grp, 0)
                    if H == 4:
                        pltpu.sync_copy(
                            co_buf,
                            co_hbm.at[kh].at[pl.ds(ebase + ch * CHUNK,
                                                   CHUNK)])
                    else:
                        @pl.when((s // 8) == c)
                        def _():
                            pltpu.sync_copy(
                                co_buf,
                                co_hbm.at[0].at[pl.ds(ebase + ch * CHUNK,
                                                      CHUNK)])
                    return 0

                lax.fori_loop(0, NCH, chunk_body, 0)
                plsc.subcore_barrier()

    return k(*eb1, *eb2)


def _sc_feat(src1, dst1, co1, hf1, src2, dst2, co2, hf2, H, Np, NPP):
    EP = src1.shape[0]
    EC = EP // NS if H == 4 else EP // NW
    T = 64
    W = 2 * F if H == 4 else F
    NB = EC // T
    ROWS = NPP // NS
    ZR = 8
    mesh = plsc.VectorSubcoreMesh(**_MESH)
    sds = jax.ShapeDtypeStruct((NC * NPP, W), jnp.float32)

    scratch = [
        pltpu.VMEM((EC,), jnp.int32),
        pltpu.VMEM((T,), jnp.int32),
        pltpu.VMEM((T,), jnp.int32),
        pltpu.VMEM((T,), jnp.float32),
        pltpu.VMEM((T,), jnp.float32),
        pltpu.VMEM((T,), jnp.int32),
        pltpu.VMEM((T,), jnp.int32),
        pltpu.VMEM((L,), jnp.int32),
        pltpu.VMEM((L,), jnp.int32),
        pltpu.VMEM((L,), jnp.int32),
        pltpu.VMEM((L,), jnp.int32),
        pltpu.VMEM((L,), jnp.int32),
        pltpu.VMEM((L,), jnp.int32),
        pltpu.VMEM((L,), jnp.int32),
        pltpu.VMEM((L,), jnp.int32),
        pltpu.VMEM((T, 2 * F), jnp.float32),
        pltpu.VMEM((T, 2 * F), jnp.float32),
        pltpu.VMEM((ZR, W), jnp.float32),
        pltpu.SemaphoreType.DMA,
        pltpu.SemaphoreType.DMA,
        pltpu.SemaphoreType.DMA,
        pltpu.SemaphoreType.DMA,
        pltpu.VMEM_SHARED((NPP, W), jnp.float32),
    ]
    if H == 4:
        scratch += [
            pltpu.VMEM((T,), jnp.float32),
            pltpu.VMEM((T,), jnp.float32),
        ]
    else:
        scratch += [
            pltpu.VMEM((T, F), jnp.float32),
            pltpu.VMEM((T, F), jnp.float32),
        ]

    @functools.partial(
        pl.kernel,
        out_type=(sds, sds),
        mesh=mesh,
        compiler_params=_CPARAMS,
        scratch_types=scratch,
    )
    def k(s1, d1, c1, h1, s2, d2, c2, h2, o1, o2,
          src_b, dst_s0, dst_s1, co0_s0, co0_s1, gidx0, gidx1,
          s0a, s0b, s0c, s0d, s1a, s1b, s1c, s1d,
          gbuf0, gbuf1, zbuf, semg0, semg1, semi0, semi1, outacc, *rest):
        sidx0 = [s0a, s0b, s0c, s0d]
        sidx1 = [s1a, s1b, s1c, s1d]
        if H == 4:
            coA_s0, coA_s1 = rest
            sbuf0 = sbuf1 = None
        else:
            sbuf0, sbuf1 = rest
            coA_s0 = coA_s1 = None
        c = lax.axis_index("c")
        s = lax.axis_index("s")
        ebase = s * EC if H == 4 else (c * NS + s) * EC
        for r in range(ZR):
            for f in range(W // L):
                zbuf[r, pl.ds(f * L, L)] = jnp.zeros((L,), jnp.float32)

        for (src_hbm, dst_hbm, co_hbm, h_hbm, out_hbm) in (
                (s1, d1, c1, h1, o1), (s2, d2, c2, h2, o2)):
            pltpu.sync_copy(src_hbm.at[pl.ds(ebase, EC)], src_b)

            def zb(i, _):
                pltpu.sync_copy(zbuf, outacc.at[pl.ds(s * ROWS + i * ZR, ZR)])
                return 0

            lax.fori_loop(0, ROWS // ZR, zb, 0)
            gbase = c * Np if H == 4 else 0
            k0 = 2 * c
            plsc.subcore_barrier()

            def prep_fire(b, dst_s, co0_s, coA_s, gidx, gbuf, semg, semi):
                for t in range(T // L):
                    sv = src_b[pl.ds(b * T + t * L, L)]
                    gidx[pl.ds(t * L, L)] = sv + gbase
                pltpu.async_copy(h_hbm.at[gidx], gbuf, semg)
                eoff = ebase + b * T
                pltpu.async_copy(dst_hbm.at[pl.ds(eoff, T)], dst_s, semi)
                if H == 4:
                    pltpu.async_copy(co_hbm.at[k0].at[pl.ds(eoff, T)],
                                     co0_s, semi)
                    pltpu.async_copy(co_hbm.at[k0 + 1].at[pl.ds(eoff, T)],
                                     coA_s, semi)
                else:
                    pltpu.async_copy(co_hbm.at[0].at[pl.ds(eoff, T)],
                                     co0_s, semi)

            def consume(b, dst_s, co0_s, coA_s, sidx, gbuf, sbuf, semg,
                        semi):
                pltpu.make_async_copy(h_hbm.at[pl.ds(0, T)], gbuf,
                                      semg).wait()
                pltpu.make_async_copy(dst_hbm.at[pl.ds(0, T)], dst_s,
                                      semi).wait()
                pltpu.make_async_copy(dst_hbm.at[pl.ds(0, T)], co0_s,
                                      semi).wait()
                if H == 4:
                    pltpu.make_async_copy(dst_hbm.at[pl.ds(0, T)], coA_s,
                                          semi).wait()
                for t in range(T // L):
                    sidx[t][pl.ds(0, L)] = dst_s[pl.ds(t * L, L)]
                    co0 = co0_s[pl.ds(t * L, L)]
                    coA = coA_s[pl.ds(t * L, L)] if H == 4 else None
                    for r16 in range(L):
                        r = t * L + r16
                        if H == 4:
                            for f in range(8):
                                cr = co0[r16] if f < 4 else coA[r16]
                                gbuf[r, pl.ds(f * L, L)] = (
                                    gbuf[r, pl.ds(f * L, L)] * cr)
                        else:
                            cr = co0[r16]
                            for f in range(4):
                                sbuf[r, pl.ds(f * L, L)] = (
                                    gbuf[r, pl.ds(f * L, L)] * cr)
                src_buf = gbuf if H == 4 else sbuf
                for t in range(T // L):
                    pltpu.sync_copy(src_buf.at[pl.ds(t * L, L)],
                                    outacc.at[sidx[t]], add=True)

            prep_fire(0, dst_s0, co0_s0, coA_s0, gidx0, gbuf0, semg0, semi0)

            def pair(pp, _):
                b0 = 2 * pp
                prep_fire(b0 + 1, dst_s1, co0_s1, coA_s1, gidx1, gbuf1,
                          semg1, semi1)
                consume(b0, dst_s0, co0_s0, coA_s0, sidx0, gbuf0, sbuf0,
                        semg0, semi0)

                @pl.when(pp < NB // 2 - 1)
                def _():
                    prep_fire(b0 + 2, dst_s0, co0_s0, coA_s0, gidx0, gbuf0,
                              semg0, semi0)

                consume(b0 + 1, dst_s1, co0_s1, coA_s1, sidx1, gbuf1, sbuf1,
                        semg1, semi1)
                return 0

            lax.fori_loop(0, NB // 2, pair, 0)
            plsc.subcore_barrier()
            pltpu.sync_copy(outacc.at[pl.ds(s * ROWS, ROWS)],
                            out_hbm.at[pl.ds(c * NPP + s * ROWS, ROWS)])
            plsc.subcore_barrier()

    return k(src1, dst1, co1, hf1, src2, dst2, co2, hf2)


def _sc_pool(of1, of2, bias, batch, B, Np, NPP):
    SEG = B // NW
    SEGC = B // NC
    NEG = -3.0e38
    mesh = plsc.VectorSubcoreMesh(**_MESH)
    sds = jax.ShapeDtypeStruct((B, F), jnp.float32)

    @functools.partial(
        pl.kernel,
        out_type=(sds, sds),
        mesh=mesh,
        compiler_params=_CPARAMS,
        scratch_types=[
            pltpu.VMEM((Np,), jnp.int32),
            pltpu.VMEM((F,), jnp.float32),
            pltpu.VMEM((8, F), jnp.float32),
            pltpu.VMEM((8, F), jnp.float32),
            pltpu.VMEM((SEG, F), jnp.float32),
            pltpu.VMEM_SHARED((SEGC, F), jnp.float32),
        ],
    )
    def k(p1_hbm, p2_hbm, bias_hbm, bat_hbm, g1_hbm, g2_hbm,
          bat_b, bias_b, pbuf, qbuf, gout, gstage):
        c = lax.axis_index("c")
        s = lax.axis_index("s")
        t0 = (c * NS + s) * SEG
        pltpu.sync_copy(bat_hbm, bat_b)
        pltpu.sync_copy(bias_hbm, bias_b)

        def cnt_body(i, accs):
            v = bat_b[pl.ds(i * L, L)]
            return tuple(
                a + jnp.where(v < t0 + j, 1, 0).astype(jnp.int32)
                for j, a in enumerate(accs))

        accs = lax.fori_loop(0, Np // L, cnt_body,
                             tuple(jnp.zeros((L,), jnp.int32)
                                   for _ in range(SEG + 1)))
        bounds = [jnp.sum(a) for a in accs]

        for (p_hbm, g_hbm) in ((p1_hbm, g1_hbm), (p2_hbm, g2_hbm)):
            for j in range(SEG):
                lo, hi = bounds[j], bounds[j + 1]
                lo8 = (lo // 8) * 8
                nch = (hi - lo8 + 7) // 8

                def chunk(ch, acc):
                    r0 = lo8 + ch * 8
                    pltpu.sync_copy(p_hbm.at[pl.ds(r0, 8)], pbuf)
                    pltpu.sync_copy(p_hbm.at[pl.ds(NPP + r0, 8)], qbuf)
                    acc = list(acc)
                    for r8 in range(8):
                        rg = r0 + r8
                        incl = (rg >= lo) & (rg < hi)
                        for f in range(F // L):
                            v = (pbuf[r8, pl.ds(f * L, L)]
                                 + qbuf[r8, pl.ds(f * L, L)]
                                 + bias_b[pl.ds(f * L, L)])
                            v = jnp.where(v > 0, v,
                                          jnp.exp(jnp.minimum(v, 0.0)) - 1.0)
                            acc[f] = jnp.maximum(
                                acc[f], jnp.where(incl, v, NEG))
                    return tuple(acc)

                acc = lax.fori_loop(
                    0, nch, chunk,
                    tuple(jnp.full((L,), NEG, jnp.float32)
                          for _ in range(F // L)))
                nonempty = hi > lo
                for f in range(F // L):
                    gout[j, pl.ds(f * L, L)] = jnp.where(
                        nonempty, acc[f], jnp.zeros((L,), jnp.float32))
            for j in range(SEG):
                pltpu.sync_copy(gout.at[j], gstage.at[s * SEG + j])
            plsc.subcore_barrier()

            @pl.when(s < SEGC // 8)
            def _():
                pltpu.sync_copy(gstage.at[pl.ds(s * 8, 8)],
                                g_hbm.at[pl.ds(c * SEGC + s * 8, 8)])

            plsc.subcore_barrier()

    return k(of1, of2, bias, batch)


def _edge_prep(ei, Np, EP, Ereal):
    loops = jnp.arange(Np, dtype=jnp.int32)
    pad = jnp.zeros((EP - Ereal,), jnp.int32)
    src = jnp.concatenate([ei[0].astype(jnp.int32), loops, pad])
    dst = jnp.concatenate([ei[1].astype(jnp.int32), loops, pad])
    return src, dst


def _transpose_pad(lg, NPP):
    Np, C = lg.shape
    bc = 2048

    def body(l_ref, o_ref):
        o_ref[...] = l_ref[...].T

    return pl.pallas_call(
        body,
        grid=(NPP // bc,),
        in_specs=[pl.BlockSpec((bc, C), lambda i: (i, 0))],
        out_specs=pl.BlockSpec((C, bc), lambda i: (0, i)),
        out_shape=jax.ShapeDtypeStruct((C, NPP), jnp.float32),
    )(lg)


def _logit_prep(hw, att_s, att_d, Np, NPP):
    H = att_s.shape[0]
    lg = _att_logits(hw, att_s, att_d)
    cm = _col_max(lg)
    C = cm[0, :H] + cm[0, H:]
    D = jnp.maximum(C, 0.2 * C)
    dvec = jnp.broadcast_to(D[:, None], (H, 16))
    lgt = _transpose_pad(lg, NPP)
    return lgt, dvec


def kernel(x1, edge_index1, x2, edge_index2, batch, gene_exp, drr, drc,
           prot_exp, params):
    p = params
    B = gene_exp.shape[0]
    Np = x1.shape[0]
    E = edge_index1.shape[1]
    Ereal = E + Np
    EP = -(-Ereal // 16384) * 16384
    NPP = -(-Np // (L * NS)) * (L * NS)

    src1, dst1 = _edge_prep(edge_index1, Np, EP, Ereal)
    src2, dst2 = _edge_prep(edge_index2, Np, EP, Ereal)

    h1w_1 = _mm(x1, p['W1'])
    h1w_2 = _mm(x2, p['W1'])
    lgt1_1, dv1_1 = _logit_prep(h1w_1, p['as1'], p['ad1'], Np, NPP)
    lgt1_2, dv1_2 = _logit_prep(h1w_2, p['as1'], p['ad1'], Np, NPP)
    eb1 = (src1, dst1, lgt1_1, dv1_1)
    eb2 = (src2, dst2, lgt1_2, dv1_2)
    co1, co2 = _sc_coef(eb1, eb2, 4, Ereal, Np, NPP)

    def pairmajor(hw):
        return (hw.reshape(Np, 2, 2 * F).transpose(1, 0, 2)
                .reshape(2 * Np, 2 * F))

    o1, o2 = _sc_feat(src1, dst1, co1, pairmajor(h1w_1),
                      src2, dst2, co2, pairmajor(h1w_2), 4, Np, NPP)

    def headcat(o):
        return (o.reshape(2, NPP, 2 * F)[:, :Np].transpose(1, 0, 2)
                .reshape(Np, 4 * F))

    h2w_1 = _mm(headcat(o1), p['W2'], in_bias=p['b1'], in_act='elu')
    h2w_2 = _mm(headcat(o2), p['W2'], in_bias=p['b1'], in_act='elu')
    lgt2_1, dv2_1 = _logit_prep(h2w_1, p['as2'], p['ad2'], Np, NPP)
    lgt2_2, dv2_2 = _logit_prep(h2w_2, p['as2'], p['ad2'], Np, NPP)
    eb1b = (src1, dst1, lgt2_1, dv2_1)
    eb2b = (src2, dst2, lgt2_2, dv2_2)
    cb1, cb2 = _sc_coef(eb1b, eb2b, 1, Ereal, Np, NPP)
    q1, q2 = _sc_feat(src1, dst1, cb1, jnp.pad(h2w_1, ((0, 0), (0, F))),
                      src2, dst2, cb2, jnp.pad(h2w_2, ((0, 0), (0, F))),
                      1, Np, NPP)

    gp1, gp2 = _sc_pool(q1, q2, p['b2'], batch.astype(jnp.int32), B, Np, NPP)
    g1 = _mm(gp1, p['Wg'], p['bg'], act='relu')
    g2 = _mm(gp2, p['Wg'], p['bg'], act='relu')

    c = _mm(gene_exp, p['Wr1'], p['br1'], act='relu', l2norm=True)
    c = _mm(c, p['Wr2'], p['br2'], act='relu')
    c = _mm(c, p['Wr3'], p['br3'], act='relu')

    pe = _mm(prot_exp.reshape(-1, 6688), p['Wp1'], p['bp1'], act='elu')
    pe = _mm(pe, p['Wp2'], p['bp2'], act='elu')
    pe = _mm(pe, p['Wp3'], p['bp3'], act='relu')

    d1 = _mm(jnp.concatenate([g1, drr], axis=1), p['Wfr'], p['bfr'],
             act='relu')
    d2 = _mm(jnp.concatenate([g2, drc], axis=1), p['Wfc'], p['bfc'],
             act='relu')

    y = jnp.concatenate([d1, d2, c, pe], axis=1)
    h = _mm(y, p['Wq1'], p['bq1'], act='relu')
    h = _mm(h, p['Wq2'], p['bq2'], act='relu')
    return _mm(h, p['Wq3'], p['bq3'])
